# Initial kernel scaffold; baseline (speedup 1.0000x reference)
import jax
import jax.numpy as jnp
from jax.experimental import pallas as pl


def kernel(x, edge_index, Wl1, Wr1, b1, Wl2, Wr2, b2):
    raise NotImplementedError("write your pallas kernel here")


# speedup vs baseline: 3.8239x; 3.8239x over previous
import functools

import jax
import jax.numpy as jnp
from jax import lax
from jax.experimental import pallas as pl
from jax.experimental.pallas import tpu as pltpu
from jax.experimental.pallas import tpu_sc as plsc

N = 10000
E = 320000
D_IN = 128
D_HID = 256
D_OUT = 128

NC = 2
NS = 16
NW = NC * NS
CHUNK = 128
NP = 10240
RPT = NP // NS
NB = RPT // CHUNK

CH_TOT = 2528
E_PAD = CH_TOT * CHUNK
L1C = CH_TOT // NW
L2G = 2
L2C = CH_TOT // (NS * L2G)


def _sc_mesh():
    return plsc.VectorSubcoreMesh(core_axis_name="c", subcore_axis_name="s")


def _sc_agg1(xp, src3d, dst3d, zeros_f, ones_f, didx):
    @functools.partial(
        pl.kernel,
        mesh=_sc_mesh(),
        out_type=[
            jax.ShapeDtypeStruct((NC, NS, NB, CHUNK, D_IN), jnp.float32),
            jax.ShapeDtypeStruct((NC, NS, NB, CHUNK, D_IN), jnp.float32),
        ],
        scratch_types=[
            pltpu.VMEM((CHUNK,), jnp.int32),
            pltpu.VMEM((CHUNK,), jnp.int32),
            pltpu.VMEM((CHUNK, D_IN), jnp.float32),
            pltpu.VMEM((CHUNK, D_IN), jnp.float32),
            pltpu.VMEM((CHUNK,), jnp.int32),
            pltpu.VMEM_SHARED((NP, D_IN), jnp.float32),
            pltpu.SemaphoreType.DMA,
        ],
    )
    def k(x_hbm, src_hbm, dst_hbm, zf_hbm, of_hbm, didx_hbm,
          agg_out, deg_out, src_v, dst_v, rows_v, ones_v, didx_v,
          acc_s, sem):
        c = lax.axis_index("c")
        s = lax.axis_index("s")
        w = c * NS + s

        pltpu.sync_copy(zf_hbm, rows_v)

        def zblk(t, carry):
            pltpu.sync_copy(didx_hbm.at[s, t, 0], didx_v)
            pltpu.sync_copy(rows_v, acc_s.at[didx_v])
            return carry

        lax.fori_loop(0, NB, zblk, 0)
        plsc.subcore_barrier()

        def body(j, carry):
            pltpu.sync_copy(src_hbm.at[w, j, 0], src_v)
            pltpu.sync_copy(dst_hbm.at[w, j, 0], dst_v)
            pltpu.async_copy(x_hbm.at[src_v], rows_v, sem).wait()
            pltpu.sync_copy(rows_v, acc_s.at[dst_v], add=True)
            return carry

        lax.fori_loop(0, L1C, body, 0)
        plsc.subcore_barrier()

        pltpu.sync_copy(zf_hbm, ones_v)

        def dblk(t, carry):
            pltpu.sync_copy(didx_hbm.at[s, t, 0], didx_v)
            pltpu.sync_copy(acc_s.at[didx_v], rows_v)
            pltpu.sync_copy(rows_v, agg_out.at[c, s, t])
            pltpu.sync_copy(ones_v, acc_s.at[didx_v])
            return carry

        lax.fori_loop(0, NB, dblk, 0)
        pltpu.sync_copy(of_hbm, ones_v)
        plsc.subcore_barrier()

        def dbody(j, carry):
            pltpu.sync_copy(dst_hbm.at[w, j, 0], dst_v)
            pltpu.sync_copy(ones_v, acc_s.at[dst_v], add=True)
            return carry

        lax.fori_loop(0, L1C, dbody, 0)
        plsc.subcore_barrier()

        def gblk(t, carry):
            pltpu.sync_copy(didx_hbm.at[s, t, 0], didx_v)
            pltpu.sync_copy(acc_s.at[didx_v], rows_v)
            pltpu.sync_copy(rows_v, deg_out.at[c, s, t])
            return carry

        lax.fori_loop(0, NB, gblk, 0)

    return k(xp, src3d, dst3d, zeros_f, ones_f, didx)


def _sc_agg2(h2, src_stack, dst16, zeros_f, didx):
    @functools.partial(
        pl.kernel,
        mesh=_sc_mesh(),
        out_type=jax.ShapeDtypeStruct((NC, NS, NB, CHUNK, D_IN), jnp.float32),
        scratch_types=[
            pltpu.VMEM((CHUNK,), jnp.int32),
            pltpu.VMEM((CHUNK,), jnp.int32),
            pltpu.VMEM((CHUNK, D_IN), jnp.float32),
            pltpu.VMEM((CHUNK,), jnp.int32),
            pltpu.VMEM_SHARED((NP, D_IN), jnp.float32),
            pltpu.SemaphoreType.DMA,
        ],
    )
    def k(h_hbm, src_hbm, dst_hbm, zf_hbm, didx_hbm,
          agg_out, src_v, dst_v, rows_v, didx_v, acc_s, sem):
        c = lax.axis_index("c")
        s = lax.axis_index("s")

        pltpu.sync_copy(zf_hbm, rows_v)

        def zblk(t, carry):
            pltpu.sync_copy(didx_hbm.at[s, t, 0], didx_v)
            pltpu.sync_copy(rows_v, acc_s.at[didx_v])
            return carry

        lax.fori_loop(0, NB, zblk, 0)
        plsc.subcore_barrier()

        def body(j, carry):
            pltpu.sync_copy(src_hbm.at[c, s, j, 0], src_v)
            pltpu.sync_copy(dst_hbm.at[s, j, 0], dst_v)
            pltpu.async_copy(h_hbm.at[src_v], rows_v, sem).wait()
            pltpu.sync_copy(rows_v, acc_s.at[dst_v], add=True)
            return carry

        lax.fori_loop(0, L2G * L2C, body, 0)
        plsc.subcore_barrier()

        def dblk(t, carry):
            pltpu.sync_copy(didx_hbm.at[s, t, 0], didx_v)
            pltpu.sync_copy(acc_s.at[didx_v], rows_v)
            pltpu.sync_copy(rows_v, agg_out.at[c, s, t])
            return carry

        lax.fori_loop(0, NB, dblk, 0)

    return k(h2, src_stack, dst16, zeros_f, didx)


_RB = 1024


def _tc_layer1(agg_part, deg_part, xp, Wl1, Wr1, b1):
    def body(ap_ref, dp_ref, x_ref, wl_ref, wr_ref, b_ref, h_ref):
        agg = ap_ref[0] + ap_ref[1]
        deg = dp_ref[0, :, 0:1] + dp_ref[1, :, 0:1]
        mean = agg / jnp.maximum(deg, 1.0)
        hh = (jnp.dot(mean, wl_ref[...], preferred_element_type=jnp.float32)
              + b_ref[...]
              + jnp.dot(x_ref[...], wr_ref[...],
                        preferred_element_type=jnp.float32))
        hh = jnp.maximum(hh, 0.0)
        h_ref[0] = hh[:, :D_IN]
        h_ref[1] = hh[:, D_IN:]

    grid = (NP // _RB,)
    return pl.pallas_call(
        body,
        grid=grid,
        in_specs=[
            pl.BlockSpec((NC, _RB, D_IN), lambda i: (0, i, 0)),
            pl.BlockSpec((NC, _RB, D_IN), lambda i: (0, i, 0)),
            pl.BlockSpec((_RB, D_IN), lambda i: (i, 0)),
            pl.BlockSpec((D_IN, D_HID), lambda i: (0, 0)),
            pl.BlockSpec((D_IN, D_HID), lambda i: (0, 0)),
            pl.BlockSpec((1, D_HID), lambda i: (0, 0)),
        ],
        out_specs=pl.BlockSpec((NC, _RB, D_IN), lambda i: (0, i, 0)),
        out_shape=jax.ShapeDtypeStruct((NC, NP, D_IN), jnp.float32),
    )(agg_part, deg_part, xp, Wl1, Wr1, b1)


def _tc_layer2(agg2_part, deg_part, h_parts, Wl2, Wr2, b2):
    def body(ap_ref, dp_ref, h_ref, wl_ref, wr_ref, b_ref, o_ref):
        deg = jnp.maximum(dp_ref[0, :, 0:1] + dp_ref[1, :, 0:1], 1.0)
        m0 = ap_ref[0] / deg
        m1 = ap_ref[1] / deg
        acc = (jnp.dot(m0, wl_ref[:D_IN, :], preferred_element_type=jnp.float32)
               + jnp.dot(m1, wl_ref[D_IN:, :], preferred_element_type=jnp.float32)
               + b_ref[...]
               + jnp.dot(h_ref[0], wr_ref[:D_IN, :],
                         preferred_element_type=jnp.float32)
               + jnp.dot(h_ref[1], wr_ref[D_IN:, :],
                         preferred_element_type=jnp.float32))
        o_ref[...] = jax.nn.sigmoid(acc)

    grid = (NP // _RB,)
    return pl.pallas_call(
        body,
        grid=grid,
        in_specs=[
            pl.BlockSpec((NC, _RB, D_IN), lambda i: (0, i, 0)),
            pl.BlockSpec((NC, _RB, D_IN), lambda i: (0, i, 0)),
            pl.BlockSpec((NC, _RB, D_IN), lambda i: (0, i, 0)),
            pl.BlockSpec((D_HID, D_OUT), lambda i: (0, 0)),
            pl.BlockSpec((D_HID, D_OUT), lambda i: (0, 0)),
            pl.BlockSpec((1, D_OUT), lambda i: (0, 0)),
        ],
        out_specs=pl.BlockSpec((_RB, D_OUT), lambda i: (i, 0)),
        out_shape=jax.ShapeDtypeStruct((NP, D_OUT), jnp.float32),
    )(agg2_part, deg_part, h_parts, Wl2, Wr2, b2)


def kernel(x, edge_index, Wl1, Wr1, b1, Wl2, Wr2, b2):
    src = edge_index[0].astype(jnp.int32)
    dst = edge_index[1].astype(jnp.int32)
    pad = E_PAD - E
    src = jnp.concatenate([src, jnp.zeros((pad,), jnp.int32)])
    dst = jnp.concatenate([dst, jnp.full((pad,), N, jnp.int32)])

    src3d = src.reshape(NW, L1C, 1, CHUNK)
    dst3d = dst.reshape(NW, L1C, 1, CHUNK)
    src16 = src.reshape(NS, L2G * L2C, 1, CHUNK)
    dst16 = dst.reshape(NS, L2G * L2C, 1, CHUNK)
    src_stack = jnp.stack([src16, src16 + NP])

    xp = jnp.zeros((NP, D_IN), jnp.float32).at[:N].set(x)
    zeros_f = jnp.zeros((CHUNK, D_IN), jnp.float32)
    ones_f = jnp.ones((CHUNK, D_IN), jnp.float32)
    didx = jnp.arange(NP, dtype=jnp.int32).reshape(NS, NB, 1, CHUNK)

    agg1_part, deg_part = _sc_agg1(xp, src3d, dst3d, zeros_f, ones_f, didx)
    agg1_part = agg1_part.reshape(NC, NP, D_IN)
    deg_part = deg_part.reshape(NC, NP, D_IN)
    h_parts = _tc_layer1(agg1_part, deg_part, xp, Wl1, Wr1,
                         b1.reshape(1, D_HID))
    h2 = h_parts.reshape(NC * NP, D_IN)
    agg2_part = _sc_agg2(h2, src_stack, dst16, zeros_f, didx)
    agg2_part = agg2_part.reshape(NC, NP, D_IN)
    out = _tc_layer2(agg2_part, deg_part, h_parts, Wl2, Wr2,
                     b2.reshape(1, D_OUT))
    return out[:N]
